# baseline (speedup 1.0000x reference)
import functools

import jax
import jax.numpy as jnp
import numpy as np
from jax import lax
from jax.experimental import pallas as pl
from jax.experimental.pallas import tpu as pltpu
from jax.experimental.pallas import tpu_sc as plsc

D_IN = 2048
D_OUT = 2048
N_FEAT = 16384
TOP_K = 32
N_TOK = 4096


PREP_TB = 1024


def _prep_body(x_ref, xp_ref, norm_ref):
    xb = x_ref[...]
    n = jnp.sqrt(jnp.sum(xb * xb, axis=1, keepdims=True))
    xp_ref[...] = xb / (n + 1e-8)
    norm_ref[...] = n


def _prep(x):
    nt = N_TOK // PREP_TB
    return pl.pallas_call(
        _prep_body,
        grid=(nt,),
        in_specs=[pl.BlockSpec((PREP_TB, D_IN), lambda t: (t, 0))],
        out_specs=[
            pl.BlockSpec((PREP_TB, D_IN), lambda t: (t, 0)),
            pl.BlockSpec((PREP_TB, 1), lambda t: (t, 0)),
        ],
        out_shape=[
            jax.ShapeDtypeStruct((N_TOK, D_IN), jnp.float32),
            jax.ShapeDtypeStruct((N_TOK, 1), jnp.float32),
        ],
    )(x)


ENC_TB = 512
ENC_FB = 1024


def _encode_body(xp_ref, w_ref, b_ref, pre_ref):
    acc = jax.lax.dot_general(
        xp_ref[...], w_ref[...], (((1,), (1,)), ((), ())),
        preferred_element_type=jnp.float32)
    pre_ref[...] = acc + b_ref[...]


def _encode(xp, W_enc, b_enc2d):
    nf = N_FEAT // ENC_FB
    nt = N_TOK // ENC_TB
    return pl.pallas_call(
        _encode_body,
        grid=(nf, nt),
        in_specs=[
            pl.BlockSpec((ENC_TB, D_IN), lambda f, t: (t, 0)),
            pl.BlockSpec((ENC_FB, D_IN), lambda f, t: (f, 0)),
            pl.BlockSpec((1, ENC_FB), lambda f, t: (0, f)),
        ],
        out_specs=pl.BlockSpec((ENC_TB, ENC_FB), lambda f, t: (t, f)),
        out_shape=jax.ShapeDtypeStruct((N_TOK, N_FEAT), jnp.float32),
    )(xp, W_enc, b_enc2d)


L = 16
NV = N_FEAT // L
NW = 32
ROWS_PER_W = N_TOK // NW
CAND_MAX = 2048
NEG = np.float32(-3.4e38)
MASK31 = np.int32(0x7FFFFFFF)
INT_MIN = np.int32(-2147483648)


def _keys(v):
    u = plsc.bitcast(v, jnp.int32)
    return jnp.where(u < 0, u ^ MASK31, u)


def _bitsearch(count_ge):
    t0 = jnp.where(count_ge(jnp.int32(0)) >= TOP_K, jnp.int32(0), INT_MIN)

    def step(i, t):
        cand = t | lax.shift_left(jnp.int32(1), jnp.int32(30) - i)
        return jnp.where(count_ge(cand) >= TOP_K, cand, t)

    return lax.fori_loop(0, 31, step, t0)


UN = 8


RB = 4


def _sc_select_body(pre_hbm, hid_hbm, rows_ref, cand_ref):
    wid = lax.axis_index("s") * 2 + lax.axis_index("c")

    def do_batch(ib, carry):
        row0 = wid * ROWS_PER_W + ib * RB
        pltpu.sync_copy(pre_hbm.at[pl.ds(row0, RB)], rows_ref)
        for rr in range(RB):
            _one_row(rows_ref, rr, cand_ref)
        pltpu.sync_copy(rows_ref, hid_hbm.at[pl.ds(row0, RB)])
        return carry

    lax.fori_loop(0, ROWS_PER_W // RB, do_batch, 0)


def _one_row(rows_ref, rr, cand_ref):
    if True:
        def p1(j, c):
            r1s, r2s = c
            out1, out2 = [], []
            for u in range(UN):
                v = rows_ref[rr, pl.ds((j * UN + u) * L, L)]
                hi = jnp.maximum(r1s[u], v)
                lo = jnp.minimum(r1s[u], v)
                out1.append(hi)
                out2.append(jnp.maximum(r2s[u], lo))
            return tuple(out1), tuple(out2)

        neg = jnp.full((L,), NEG, jnp.float32)
        r1s, r2s = lax.fori_loop(
            0, NV // UN, p1,
            (tuple(neg for _ in range(UN)), tuple(neg for _ in range(UN))))
        big1 = r1s[0]
        for u in range(1, UN):
            big1 = jnp.maximum(big1, r1s[u])
        big2 = jnp.full((L,), NEG, jnp.float32)
        for u in range(UN):
            big2 = jnp.maximum(
                big2, jnp.where(r1s[u] == big1, r2s[u], r1s[u]))
        tau0v = jnp.full((L,), jnp.min(big2), jnp.float32)

        def p2(j, c):
            off, n = c
            vs = [rows_ref[rr, pl.ds((j * UN + u) * L, L)] for u in range(UN)]
            ms = [v >= tau0v for v in vs]
            mi = ms[0].astype(jnp.int32)
            for u in range(1, UN):
                mi = mi + ms[u].astype(jnp.int32)
            g = jnp.sum(mi)

            def hit():
                o = off
                for u in range(UN):
                    plsc.store_compressed(
                        cand_ref.at[pl.ds(o, L)], vs[u], mask=ms[u])
                    o = jnp.minimum(
                        o + jnp.sum(ms[u].astype(jnp.int32)), CAND_MAX)
                return o

            off = lax.cond(g > 0, hit, lambda: off)
            return off, n + g

        off, n = lax.fori_loop(0, NV // UN, p2, (jnp.int32(0), jnp.int32(0)))
        cand_ref[pl.ds(off, L)] = jnp.full((L,), NEG)

        def count_maker(ref, nv, to_key):
            def count_ge(t):
                def s(j, acc):
                    kv = to_key(ref[pl.ds(j * L, L)])
                    return acc + (kv >= t).astype(jnp.int32)
                av = lax.fori_loop(0, nv, s, jnp.zeros((L,), jnp.int32))
                return jnp.sum(av)
            return count_ge

        def search_cand():
            nv = (n + (L - 1)) // L
            def conv(j, _):
                cand_ref[pl.ds(j * L, L)] = plsc.bitcast(
                    _keys(cand_ref[pl.ds(j * L, L)]), jnp.float32)
                return 0
            lax.fori_loop(0, nv, conv, 0)
            kref = cand_ref
            def to_key(v):
                return plsc.bitcast(v, jnp.int32)
            return _bitsearch(count_maker(kref, nv, to_key))

        def search_row():
            def count_ge(t):
                def sr(j, acc):
                    kv = _keys(rows_ref[rr, pl.ds(j * L, L)])
                    return acc + (kv >= t).astype(jnp.int32)
                av = lax.fori_loop(0, NV, sr, jnp.zeros((L,), jnp.int32))
                return jnp.sum(av)
            return _bitsearch(count_ge)

        tkey = lax.cond(n <= CAND_MAX, search_cand, search_row)

        tkv = jnp.full((L,), tkey, jnp.int32)
        tauf = jnp.min(plsc.bitcast(
            jnp.where(tkv < 0, tkv ^ MASK31, tkv), jnp.float32))

        def p3(j, _):
            for u in range(UN):
                v = rows_ref[rr, pl.ds((j * UN + u) * L, L)]
                rows_ref[rr, pl.ds((j * UN + u) * L, L)] = jnp.where(
                    v >= tauf, jnp.maximum(v, 0.0), 0.0)
            return 0

        lax.fori_loop(0, NV // UN, p3, 0)


def _select(pre_acts):
    mesh = plsc.VectorSubcoreMesh(core_axis_name="c", subcore_axis_name="s")
    hid_flat = pl.kernel(
        _sc_select_body,
        mesh=mesh,
        out_type=jax.ShapeDtypeStruct((N_TOK, N_FEAT), jnp.float32),
        scratch_types=[
            pltpu.VMEM((RB, N_FEAT), jnp.float32),
            pltpu.VMEM((CAND_MAX + L,), jnp.float32),
        ],
        compiler_params=pltpu.CompilerParams(needs_layout_passes=False),
    )(pre_acts)
    return hid_flat


SEL_TB = 128


def _select_tc_body(pre_ref, hid_ref):
    a = pre_ref[...]
    u = jax.lax.bitcast_convert_type(a, jnp.int32)
    kk = jnp.where(u < 0, u ^ jnp.int32(0x7FFFFFFF), u)

    def cnt_ge(t):
        return jnp.sum((kk >= t).astype(jnp.int32), axis=1, keepdims=True)

    int_min = jnp.int32(-2147483648)
    t0 = jnp.where(cnt_ge(jnp.zeros((SEL_TB, 1), jnp.int32)) >= TOP_K,
                   jnp.int32(0), int_min)

    def step(i, t):
        bit = jax.lax.shift_left(jnp.int32(1), jnp.int32(30) - i)
        cand = t | bit
        return jnp.where(cnt_ge(cand) >= TOP_K, cand, t)

    t = jax.lax.fori_loop(0, 31, step, t0)
    mask = kk >= t
    hid_ref[...] = jnp.where(mask, jnp.maximum(a, 0.0), 0.0)


def _select_tc(pre_acts):
    nt = N_TOK // SEL_TB
    return pl.pallas_call(
        _select_tc_body,
        grid=(nt,),
        in_specs=[pl.BlockSpec((SEL_TB, N_FEAT), lambda t: (t, 0))],
        out_specs=pl.BlockSpec((SEL_TB, N_FEAT), lambda t: (t, 0)),
        out_shape=jax.ShapeDtypeStruct((N_TOK, N_FEAT), jnp.float32),
    )(pre_acts)


DEC_TB = 1024
DEC_KB = 1024


def _decode_body(hid_ref, w_ref, b_ref, norm_ref, out_ref, *, nk):
    k = pl.program_id(1)

    @pl.when(k == 0)
    def _():
        out_ref[...] = jnp.zeros_like(out_ref)

    out_ref[...] += jax.lax.dot_general(
        hid_ref[...], w_ref[...], (((1,), (1,)), ((), ())),
        preferred_element_type=jnp.float32)

    @pl.when(k == nk - 1)
    def _():
        out_ref[...] = (out_ref[...] + b_ref[...]) * norm_ref[...]


def _decode(hidden, W_dec, b_dec2d, x_norm):
    nt = N_TOK // DEC_TB
    nk = N_FEAT // DEC_KB
    return pl.pallas_call(
        functools.partial(_decode_body, nk=nk),
        grid=(nt, nk),
        in_specs=[
            pl.BlockSpec((DEC_TB, DEC_KB), lambda t, k: (t, k)),
            pl.BlockSpec((D_OUT, DEC_KB), lambda t, k: (0, k)),
            pl.BlockSpec((1, D_OUT), lambda t, k: (0, 0)),
            pl.BlockSpec((DEC_TB, 1), lambda t, k: (t, 0)),
        ],
        out_specs=pl.BlockSpec((DEC_TB, D_OUT), lambda t, k: (t, 0)),
        out_shape=jax.ShapeDtypeStruct((N_TOK, D_OUT), jnp.float32),
    )(hidden, W_dec, b_dec2d, x_norm)


def kernel(x, W_enc, b_enc, W_dec, b_dec):
    x_proc, x_norm = _prep(x)
    pre_acts = _encode(x_proc, W_enc, b_enc.reshape(1, N_FEAT))
    hidden = _select(pre_acts)
    out = _decode(hidden, W_dec, b_dec.reshape(1, D_OUT), x_norm)
    auxiliary_loss = jnp.zeros((), jnp.float32)
    return (out, hidden, pre_acts, auxiliary_loss)


# speedup vs baseline: 1.1212x; 1.1212x over previous
import functools

import jax
import jax.numpy as jnp
import numpy as np
from jax import lax
from jax.experimental import pallas as pl
from jax.experimental.pallas import tpu as pltpu
from jax.experimental.pallas import tpu_sc as plsc

D_IN = 2048
D_OUT = 2048
N_FEAT = 16384
TOP_K = 32
N_TOK = 4096


PREP_TB = 1024


def _prep_body(x_ref, xp_ref, norm_ref):
    xb = x_ref[...]
    n = jnp.sqrt(jnp.sum(xb * xb, axis=1, keepdims=True))
    xp_ref[...] = xb / (n + 1e-8)
    norm_ref[...] = n


def _prep(x):
    nt = N_TOK // PREP_TB
    return pl.pallas_call(
        _prep_body,
        grid=(nt,),
        in_specs=[pl.BlockSpec((PREP_TB, D_IN), lambda t: (t, 0))],
        out_specs=[
            pl.BlockSpec((PREP_TB, D_IN), lambda t: (t, 0)),
            pl.BlockSpec((PREP_TB, 1), lambda t: (t, 0)),
        ],
        out_shape=[
            jax.ShapeDtypeStruct((N_TOK, D_IN), jnp.float32),
            jax.ShapeDtypeStruct((N_TOK, 1), jnp.float32),
        ],
    )(x)


ENC_TB = 256
ENC_FB = 2048


def _encode_body(xp_ref, w_ref, b_ref, pre_ref):
    acc = jax.lax.dot_general(
        xp_ref[...], w_ref[...], (((1,), (1,)), ((), ())),
        preferred_element_type=jnp.float32)
    pre_ref[...] = acc + b_ref[...]


def _encode(xp, W_enc, b_enc2d):
    nf = N_FEAT // ENC_FB
    nt = N_TOK // ENC_TB
    return pl.pallas_call(
        _encode_body,
        grid=(nf, nt),
        in_specs=[
            pl.BlockSpec((ENC_TB, D_IN), lambda f, t: (t, 0)),
            pl.BlockSpec((ENC_FB, D_IN), lambda f, t: (f, 0)),
            pl.BlockSpec((1, ENC_FB), lambda f, t: (0, f)),
        ],
        out_specs=pl.BlockSpec((ENC_TB, ENC_FB), lambda f, t: (t, f)),
        out_shape=jax.ShapeDtypeStruct((N_TOK, N_FEAT), jnp.float32),
    )(xp, W_enc, b_enc2d)


L = 16
NV = N_FEAT // L
NW = 32
ROWS_PER_W = N_TOK // NW
CAND_MAX = 2048
NEG = np.float32(-3.4e38)
MASK31 = np.int32(0x7FFFFFFF)
INT_MIN = np.int32(-2147483648)


def _keys(v):
    u = plsc.bitcast(v, jnp.int32)
    return jnp.where(u < 0, u ^ MASK31, u)


def _bitsearch(count_ge):
    t0 = jnp.where(count_ge(jnp.int32(0)) >= TOP_K, jnp.int32(0), INT_MIN)

    def step(i, t):
        cand = t | lax.shift_left(jnp.int32(1), jnp.int32(30) - i)
        return jnp.where(count_ge(cand) >= TOP_K, cand, t)

    return lax.fori_loop(0, 31, step, t0)


UN = 8


RB = 4


def _sc_select_body(pre_hbm, hid_hbm, rows_ref, cand_ref):
    wid = lax.axis_index("s") * 2 + lax.axis_index("c")

    def do_batch(ib, carry):
        row0 = wid * ROWS_PER_W + ib * RB
        pltpu.sync_copy(pre_hbm.at[pl.ds(row0, RB)], rows_ref)
        for rr in range(RB):
            _one_row(rows_ref, rr, cand_ref)
        pltpu.sync_copy(rows_ref, hid_hbm.at[pl.ds(row0, RB)])
        return carry

    lax.fori_loop(0, ROWS_PER_W // RB, do_batch, 0)


def _one_row(rows_ref, rr, cand_ref):
    if True:
        def p1(j, c):
            r1s, r2s = c
            out1, out2 = [], []
            for u in range(UN):
                v = rows_ref[rr, pl.ds((j * UN + u) * L, L)]
                hi = jnp.maximum(r1s[u], v)
                lo = jnp.minimum(r1s[u], v)
                out1.append(hi)
                out2.append(jnp.maximum(r2s[u], lo))
            return tuple(out1), tuple(out2)

        neg = jnp.full((L,), NEG, jnp.float32)
        r1s, r2s = lax.fori_loop(
            0, NV // UN, p1,
            (tuple(neg for _ in range(UN)), tuple(neg for _ in range(UN))))
        big1 = r1s[0]
        for u in range(1, UN):
            big1 = jnp.maximum(big1, r1s[u])
        big2 = jnp.full((L,), NEG, jnp.float32)
        for u in range(UN):
            big2 = jnp.maximum(
                big2, jnp.where(r1s[u] == big1, r2s[u], r1s[u]))
        tau0v = jnp.full((L,), jnp.min(big2), jnp.float32)

        def p2(j, c):
            off, n = c
            vs = [rows_ref[rr, pl.ds((j * UN + u) * L, L)] for u in range(UN)]
            ms = [v >= tau0v for v in vs]
            cs = [jnp.sum(m.astype(jnp.int32)) for m in ms]
            for u in range(UN):
                plsc.store_compressed(
                    cand_ref.at[pl.ds(off, L)], vs[u], mask=ms[u])
                off = jnp.minimum(off + cs[u], CAND_MAX)
                n = n + cs[u]
            return off, n

        off, n = lax.fori_loop(0, NV // UN, p2, (jnp.int32(0), jnp.int32(0)))
        cand_ref[pl.ds(off, L)] = jnp.full((L,), NEG)

        def count_maker(ref, nv, to_key):
            def count_ge(t):
                def s(j, acc):
                    kv = to_key(ref[pl.ds(j * L, L)])
                    return acc + (kv >= t).astype(jnp.int32)
                av = lax.fori_loop(0, nv, s, jnp.zeros((L,), jnp.int32))
                return jnp.sum(av)
            return count_ge

        def search_cand():
            nv = (n + (L - 1)) // L
            def conv(j, _):
                cand_ref[pl.ds(j * L, L)] = plsc.bitcast(
                    _keys(cand_ref[pl.ds(j * L, L)]), jnp.float32)
                return 0
            lax.fori_loop(0, nv, conv, 0)
            kref = cand_ref
            def to_key(v):
                return plsc.bitcast(v, jnp.int32)
            return _bitsearch(count_maker(kref, nv, to_key))

        def search_row():
            def count_ge(t):
                def sr(j, acc):
                    kv = _keys(rows_ref[rr, pl.ds(j * L, L)])
                    return acc + (kv >= t).astype(jnp.int32)
                av = lax.fori_loop(0, NV, sr, jnp.zeros((L,), jnp.int32))
                return jnp.sum(av)
            return _bitsearch(count_ge)

        tkey = lax.cond(n <= CAND_MAX, search_cand, search_row)

        tkv = jnp.full((L,), tkey, jnp.int32)
        tauf = jnp.min(plsc.bitcast(
            jnp.where(tkv < 0, tkv ^ MASK31, tkv), jnp.float32))

        def p3(j, _):
            for u in range(UN):
                v = rows_ref[rr, pl.ds((j * UN + u) * L, L)]
                rows_ref[rr, pl.ds((j * UN + u) * L, L)] = jnp.where(
                    v >= tauf, jnp.maximum(v, 0.0), 0.0)
            return 0

        lax.fori_loop(0, NV // UN, p3, 0)


def _select(pre_acts):
    mesh = plsc.VectorSubcoreMesh(core_axis_name="c", subcore_axis_name="s")
    hid_flat = pl.kernel(
        _sc_select_body,
        mesh=mesh,
        out_type=jax.ShapeDtypeStruct((N_TOK, N_FEAT), jnp.float32),
        scratch_types=[
            pltpu.VMEM((RB, N_FEAT), jnp.float32),
            pltpu.VMEM((CAND_MAX + L,), jnp.float32),
        ],
        compiler_params=pltpu.CompilerParams(needs_layout_passes=False),
    )(pre_acts)
    return hid_flat


SEL_TB = 128


def _select_tc_body(pre_ref, hid_ref):
    a = pre_ref[...]
    u = jax.lax.bitcast_convert_type(a, jnp.int32)
    kk = jnp.where(u < 0, u ^ jnp.int32(0x7FFFFFFF), u)

    def cnt_ge(t):
        return jnp.sum((kk >= t).astype(jnp.int32), axis=1, keepdims=True)

    int_min = jnp.int32(-2147483648)
    t0 = jnp.where(cnt_ge(jnp.zeros((SEL_TB, 1), jnp.int32)) >= TOP_K,
                   jnp.int32(0), int_min)

    def step(i, t):
        bit = jax.lax.shift_left(jnp.int32(1), jnp.int32(30) - i)
        cand = t | bit
        return jnp.where(cnt_ge(cand) >= TOP_K, cand, t)

    t = jax.lax.fori_loop(0, 31, step, t0)
    mask = kk >= t
    hid_ref[...] = jnp.where(mask, jnp.maximum(a, 0.0), 0.0)


def _select_tc(pre_acts):
    nt = N_TOK // SEL_TB
    return pl.pallas_call(
        _select_tc_body,
        grid=(nt,),
        in_specs=[pl.BlockSpec((SEL_TB, N_FEAT), lambda t: (t, 0))],
        out_specs=pl.BlockSpec((SEL_TB, N_FEAT), lambda t: (t, 0)),
        out_shape=jax.ShapeDtypeStruct((N_TOK, N_FEAT), jnp.float32),
    )(pre_acts)


DEC_TB = 1024
DEC_KB = 1024


def _decode_body(hid_ref, w_ref, b_ref, norm_ref, out_ref, *, nk):
    k = pl.program_id(1)

    @pl.when(k == 0)
    def _():
        out_ref[...] = jnp.zeros_like(out_ref)

    out_ref[...] += jax.lax.dot_general(
        hid_ref[...], w_ref[...], (((1,), (1,)), ((), ())),
        preferred_element_type=jnp.float32)

    @pl.when(k == nk - 1)
    def _():
        out_ref[...] = (out_ref[...] + b_ref[...]) * norm_ref[...]


def _decode(hidden, W_dec, b_dec2d, x_norm):
    nt = N_TOK // DEC_TB
    nk = N_FEAT // DEC_KB
    return pl.pallas_call(
        functools.partial(_decode_body, nk=nk),
        grid=(nt, nk),
        in_specs=[
            pl.BlockSpec((DEC_TB, DEC_KB), lambda t, k: (t, k)),
            pl.BlockSpec((D_OUT, DEC_KB), lambda t, k: (0, k)),
            pl.BlockSpec((1, D_OUT), lambda t, k: (0, 0)),
            pl.BlockSpec((DEC_TB, 1), lambda t, k: (t, 0)),
        ],
        out_specs=pl.BlockSpec((DEC_TB, D_OUT), lambda t, k: (t, 0)),
        out_shape=jax.ShapeDtypeStruct((N_TOK, D_OUT), jnp.float32),
    )(hidden, W_dec, b_dec2d, x_norm)


def kernel(x, W_enc, b_enc, W_dec, b_dec):
    x_proc, x_norm = _prep(x)
    pre_acts = _encode(x_proc, W_enc, b_enc.reshape(1, N_FEAT))
    hidden = _select(pre_acts)
    out = _decode(hidden, W_dec, b_dec.reshape(1, D_OUT), x_norm)
    auxiliary_loss = jnp.zeros((), jnp.float32)
    return (out, hidden, pre_acts, auxiliary_loss)
